# Initial kernel scaffold; baseline (speedup 1.0000x reference)
import jax
import jax.numpy as jnp
from jax.experimental import pallas as pl


def kernel(fine_features, coarse_features, atom_to_coarse, global_features, W_f2c, b_f2c, g1, be1, W_c2f, b_c2f, g2, be2, W_gate, b_gate, W_gi, b_gi, g3, be3):
    raise NotImplementedError("write your pallas kernel here")


# speedup vs baseline: 3.5112x; 3.5112x over previous
import functools

import jax
import jax.numpy as jnp
from jax import lax
from jax.experimental import pallas as pl
from jax.experimental.pallas import tpu as pltpu
from jax.experimental.pallas import tpu_sc as plsc

_B, _N, _NC, _H = 8, 4096, 512, 512
_F32 = jnp.float32

_NCORES = 2
_NSUB = 16


@functools.lru_cache(maxsize=None)
def _make_gather(W):
    NW = _NCORES * _NSUB
    APT = _B * _N // NW
    CH = 32
    NCH = APT // CH
    WPB = NW // _B
    mesh = plsc.VectorSubcoreMesh(core_axis_name="c", subcore_axis_name="s",
                                  num_cores=_NCORES, num_subcores=_NSUB)

    @functools.partial(
        pl.kernel,
        out_type=jax.ShapeDtypeStruct((_B * _N // CH, CH, W), _F32),
        mesh=mesh,
        scratch_types=[
            pltpu.VMEM((CH,), jnp.int32),
            pltpu.VMEM((CH, W), _F32),
            pltpu.SemaphoreType.DMA,
        ],
    )
    def gather(tab_hbm, a2c_hbm, out_hbm, idx_v, rows_v, sem):
        c = lax.axis_index("c")
        s = lax.axis_index("s")
        wid = s * _NCORES + c
        base = jnp.full((16,), wid // WPB * _NC, jnp.int32)

        def body(ch, _):
            blk = wid * NCH + ch
            pltpu.sync_copy(a2c_hbm.at[blk], idx_v)
            for j in range(CH // 16):
                sl = pl.ds(j * 16, 16)
                idx_v[sl] = idx_v[sl] + base
            pltpu.async_copy(tab_hbm.at[idx_v], rows_v, sem).wait()
            pltpu.sync_copy(rows_v, out_hbm.at[blk])
            return 0
        lax.fori_loop(0, NCH, body, 0)

    return gather, CH


_G_CH = 32


def _gather_call(tab, a2c_g):
    return _make_gather(2 * _H)[0](tab, a2c_g)


def _ln(x, g, b):
    m = jnp.mean(x, axis=-1, keepdims=True)
    v = jnp.mean((x - m) ** 2, axis=-1, keepdims=True)
    return (x - m) / jnp.sqrt(v + 1e-5) * g + b


def _sigmoid(x):
    return 1.0 / (1.0 + jnp.exp(-x))


def _glob_body(g_ref, w_ref, b_ref, gw_ref, gs_ref):
    g = g_ref[0]
    gw_ref[0] = jnp.dot(g, w_ref[...], preferred_element_type=_F32) + b_ref[...]
    gs_ref[0] = jnp.sum(g, axis=0, keepdims=True)


def _seg_mean_body(f_ref, a_ref, out_ref):
    idx = a_ref[0, 0]
    iota = lax.broadcasted_iota(jnp.int32, (_N, _NC), 1)
    hot = (idx[:, None] == iota)
    cnt = jnp.sum(hot.astype(_F32), axis=0)
    rec = 1.0 / jnp.maximum(cnt, 1.0)
    sums = lax.dot_general(hot.astype(jnp.bfloat16),
                           f_ref[0].astype(jnp.bfloat16),
                           (((0,), (0,)), ((), ())),
                           preferred_element_type=_F32)
    out_ref[0] = sums * rec[:, None]


def _coarse_body(m_ref, co_ref, gs_ref, ws_ref, vec_ref, out_ref, tab_ref):
    b_f2c, g1, be1 = vec_ref[0:1], vec_ref[1:2], vec_ref[2:3]
    b_c2f, g2, be2 = vec_ref[3:4], vec_ref[4:5], vec_ref[5:6]
    b_gate, b_gi = vec_ref[6:7], vec_ref[7:8]
    g3, be3 = vec_ref[8:9], vec_ref[9:10]
    w_f2c, w_c2f = ws_ref[0], ws_ref[1]
    wg1, wg2, wgi1, wgi2 = ws_ref[2], ws_ref[3], ws_ref[4], ws_ref[5]

    coarse = co_ref[0]
    cff = jnp.maximum(_ln(jnp.dot(m_ref[0], w_f2c, preferred_element_type=_F32)
                          + b_f2c, g1, be1), 0.0)
    c2 = jnp.maximum(_ln(jnp.dot(coarse, w_c2f, preferred_element_type=_F32)
                         + b_c2f, g2, be2), 0.0)
    c3 = jnp.dot(c2, wg2, preferred_element_type=_F32)
    cg = _sigmoid(jnp.dot(coarse, wg1, preferred_element_type=_F32)
                  + jnp.dot(cff, wg2, preferred_element_type=_F32) + b_gate)
    cu = cg * coarse + (1.0 - cg) * cff
    gc = gs_ref[0] * (1.0 / _N)
    cwg = jnp.maximum(
        _ln(jnp.dot(cu, wgi1, preferred_element_type=_F32)
            + jnp.dot(gc, wgi2, preferred_element_type=_F32) + b_gi, g3, be3),
        0.0)
    out_ref[0] = cu + 0.1 * cwg
    tab_ref[0] = jnp.concatenate([c2, c3], axis=-1)


def _fine_body(f_ref, r_ref, gw_ref, wg1_ref, wgi1_ref, vec_ref, out_ref):
    b_gate, g3, be3 = vec_ref[0:1], vec_ref[1:2], vec_ref[2:3]
    f = f_ref[...]
    c2g = r_ref[:, :_H]
    g3row = r_ref[:, _H:]
    fg = _sigmoid(jnp.dot(f, wg1_ref[...], preferred_element_type=_F32)
                  + g3row + b_gate)
    fu = fg * f + (1.0 - fg) * c2g
    fwg = jnp.maximum(
        _ln(jnp.dot(fu, wgi1_ref[...], preferred_element_type=_F32)
            + gw_ref[...], g3, be3), 0.0)
    out_ref[...] = fu + 0.1 * fwg


def kernel(fine_features, coarse_features, atom_to_coarse, global_features,
           W_f2c, b_f2c, g1, be1, W_c2f, b_c2f, g2, be2,
           W_gate, b_gate, W_gi, b_gi, g3, be3):
    B, N, NC, H = _B, _N, _NC, _H
    f32 = _F32

    wg1, wg2 = W_gate[:H], W_gate[H:]
    wgi1, wgi2 = W_gi[:H], W_gi[H:]

    seg_mean = pl.pallas_call(
        _seg_mean_body,
        grid=(B,),
        in_specs=[
            pl.BlockSpec((1, N, H), lambda b: (b, 0, 0)),
            pl.BlockSpec((1, 1, N), lambda b: (b, 0, 0)),
        ],
        out_specs=pl.BlockSpec((1, NC, H), lambda b: (b, 0, 0)),
        out_shape=jax.ShapeDtypeStruct((B, NC, H), f32),
    )(fine_features, atom_to_coarse.reshape(B, 1, N))

    gw, gsum = pl.pallas_call(
        _glob_body,
        grid=(B,),
        in_specs=[
            pl.BlockSpec((1, N, H), lambda b: (b, 0, 0)),
            pl.BlockSpec((H, H), lambda b: (0, 0)),
            pl.BlockSpec((1, H), lambda b: (0, 0)),
        ],
        out_specs=[
            pl.BlockSpec((1, N, H), lambda b: (b, 0, 0)),
            pl.BlockSpec((1, 1, H), lambda b: (b, 0, 0)),
        ],
        out_shape=[
            jax.ShapeDtypeStruct((B, N, H), f32),
            jax.ShapeDtypeStruct((B, 1, H), f32),
        ],
    )(global_features, wgi2, b_gi.reshape(1, H))

    vecs = jnp.stack([b_f2c, g1, be1, b_c2f, g2, be2, b_gate, b_gi, g3, be3])
    ws = jnp.stack([W_f2c, W_c2f, wg1, wg2, wgi1, wgi2])
    coarse_out, tab = pl.pallas_call(
        _coarse_body,
        grid=(B,),
        in_specs=[
            pl.BlockSpec((1, NC, H), lambda b: (b, 0, 0)),
            pl.BlockSpec((1, NC, H), lambda b: (b, 0, 0)),
            pl.BlockSpec((1, 1, H), lambda b: (b, 0, 0)),
            pl.BlockSpec((6, H, H), lambda b: (0, 0, 0)),
            pl.BlockSpec((10, H), lambda b: (0, 0)),
        ],
        out_specs=[
            pl.BlockSpec((1, NC, H), lambda b: (b, 0, 0)),
            pl.BlockSpec((1, NC, 2 * H), lambda b: (b, 0, 0)),
        ],
        out_shape=[
            jax.ShapeDtypeStruct((B, NC, H), f32),
            jax.ShapeDtypeStruct((B, NC, 2 * H), f32),
        ],
    )(seg_mean, coarse_features, gsum, ws, vecs)

    a2c_g = atom_to_coarse.reshape(B * N // _G_CH, _G_CH)
    ctx = _gather_call(tab.reshape(B * NC, 2 * H), a2c_g)
    ctx = ctx.reshape(B * N, 2 * H)

    RB = 1024
    vec3 = jnp.stack([b_gate, g3, be3])
    fine_out = pl.pallas_call(
        _fine_body,
        grid=(B * N // RB,),
        in_specs=[
            pl.BlockSpec((RB, H), lambda i: (i, 0)),
            pl.BlockSpec((RB, 2 * H), lambda i: (i, 0)),
            pl.BlockSpec((RB, H), lambda i: (i, 0)),
            pl.BlockSpec((H, H), lambda i: (0, 0)),
            pl.BlockSpec((H, H), lambda i: (0, 0)),
            pl.BlockSpec((3, H), lambda i: (0, 0)),
        ],
        out_specs=pl.BlockSpec((RB, H), lambda i: (i, 0)),
        out_shape=jax.ShapeDtypeStruct((B * N, H), f32),
    )(fine_features.reshape(B * N, H), ctx, gw.reshape(B * N, H),
      wg1, wgi1, vec3)

    return fine_out.reshape(B, N, H), coarse_out


# speedup vs baseline: 3.5258x; 1.0041x over previous
import functools

import jax
import jax.numpy as jnp
from jax import lax
from jax.experimental import pallas as pl
from jax.experimental.pallas import tpu as pltpu
from jax.experimental.pallas import tpu_sc as plsc

_B, _N, _NC, _H = 8, 4096, 512, 512
_F32 = jnp.float32

_NCORES = 2
_NSUB = 16


@functools.lru_cache(maxsize=None)
def _make_gather(W):
    NW = _NCORES * _NSUB
    APT = _B * _N // NW
    CH = 32
    NCH = APT // CH
    WPB = NW // _B
    mesh = plsc.VectorSubcoreMesh(core_axis_name="c", subcore_axis_name="s",
                                  num_cores=_NCORES, num_subcores=_NSUB)

    @functools.partial(
        pl.kernel,
        out_type=jax.ShapeDtypeStruct((_B * _N // CH, CH, W), _F32),
        mesh=mesh,
        scratch_types=[
            pltpu.VMEM((CH,), jnp.int32),
            pltpu.VMEM((CH, W), _F32),
            pltpu.SemaphoreType.DMA,
        ],
    )
    def gather(tab_hbm, a2c_hbm, out_hbm, idx_v, rows_v, sem):
        c = lax.axis_index("c")
        s = lax.axis_index("s")
        wid = s * _NCORES + c
        base = jnp.full((16,), wid // WPB * _NC, jnp.int32)

        def body(ch, _):
            blk = wid * NCH + ch
            pltpu.sync_copy(a2c_hbm.at[blk], idx_v)
            for j in range(CH // 16):
                sl = pl.ds(j * 16, 16)
                idx_v[sl] = idx_v[sl] + base
            pltpu.async_copy(tab_hbm.at[idx_v], rows_v, sem).wait()
            pltpu.sync_copy(rows_v, out_hbm.at[blk])
            return 0
        lax.fori_loop(0, NCH, body, 0)

    return gather, CH


_G_CH = 32


def _gather_call(tab, a2c_g):
    return _make_gather(2 * _H)[0](tab, a2c_g)


def _ln(x, g, b):
    m = jnp.mean(x, axis=-1, keepdims=True)
    v = jnp.mean((x - m) ** 2, axis=-1, keepdims=True)
    return (x - m) / jnp.sqrt(v + 1e-5) * g + b


def _sigmoid(x):
    return 1.0 / (1.0 + jnp.exp(-x))


_BF16 = jnp.bfloat16


def _bdot(x, w):
    return jnp.dot(x.astype(_BF16), w, preferred_element_type=_F32)


def _glob_body(g_ref, w_ref, b_ref, gw_ref, gs_ref):
    g = g_ref[0]
    gw_ref[0] = _bdot(g, w_ref[...]) + b_ref[...]
    gs_ref[0] = jnp.sum(g, axis=0, keepdims=True)


def _seg_mean_body(f_ref, a_ref, out_ref):
    idx = a_ref[0, 0]
    iota = lax.broadcasted_iota(jnp.int32, (_N, _NC), 1)
    hot = (idx[:, None] == iota)
    cnt = jnp.sum(hot.astype(_F32), axis=0)
    rec = 1.0 / jnp.maximum(cnt, 1.0)
    sums = lax.dot_general(hot.astype(jnp.bfloat16),
                           f_ref[0].astype(jnp.bfloat16),
                           (((0,), (0,)), ((), ())),
                           preferred_element_type=_F32)
    out_ref[0] = sums * rec[:, None]


def _coarse_body(m_ref, co_ref, gs_ref, ws_ref, vec_ref, out_ref, tab_ref):
    b_f2c, g1, be1 = vec_ref[0:1], vec_ref[1:2], vec_ref[2:3]
    b_c2f, g2, be2 = vec_ref[3:4], vec_ref[4:5], vec_ref[5:6]
    b_gate, b_gi = vec_ref[6:7], vec_ref[7:8]
    g3, be3 = vec_ref[8:9], vec_ref[9:10]
    w_f2c, w_c2f = ws_ref[0], ws_ref[1]
    wg1, wg2, wgi1, wgi2 = ws_ref[2], ws_ref[3], ws_ref[4], ws_ref[5]

    coarse = co_ref[0]
    cff = jnp.maximum(_ln(_bdot(m_ref[0], w_f2c) + b_f2c, g1, be1), 0.0)
    c2 = jnp.maximum(_ln(_bdot(coarse, w_c2f) + b_c2f, g2, be2), 0.0)
    c3 = _bdot(c2, wg2)
    cg = _sigmoid(_bdot(coarse, wg1) + _bdot(cff, wg2) + b_gate)
    cu = cg * coarse + (1.0 - cg) * cff
    gc = gs_ref[0] * (1.0 / _N)
    cwg = jnp.maximum(
        _ln(_bdot(cu, wgi1) + _bdot(gc, wgi2) + b_gi, g3, be3), 0.0)
    out_ref[0] = cu + 0.1 * cwg
    tab_ref[0] = jnp.concatenate([c2, c3], axis=-1)


def _fine_body(f_ref, r_ref, gw_ref, wg1_ref, wgi1_ref, vec_ref, out_ref):
    b_gate, g3, be3 = vec_ref[0:1], vec_ref[1:2], vec_ref[2:3]
    f = f_ref[...]
    c2g = r_ref[:, :_H]
    g3row = r_ref[:, _H:]
    fg = _sigmoid(_bdot(f, wg1_ref[...]) + g3row + b_gate)
    fu = fg * f + (1.0 - fg) * c2g
    fwg = jnp.maximum(
        _ln(_bdot(fu, wgi1_ref[...]) + gw_ref[...], g3, be3), 0.0)
    out_ref[...] = fu + 0.1 * fwg


def kernel(fine_features, coarse_features, atom_to_coarse, global_features,
           W_f2c, b_f2c, g1, be1, W_c2f, b_c2f, g2, be2,
           W_gate, b_gate, W_gi, b_gi, g3, be3):
    B, N, NC, H = _B, _N, _NC, _H
    f32 = _F32

    wg1, wg2 = W_gate[:H], W_gate[H:]
    wgi1, wgi2 = W_gi[:H], W_gi[H:]

    seg_mean = pl.pallas_call(
        _seg_mean_body,
        grid=(B,),
        in_specs=[
            pl.BlockSpec((1, N, H), lambda b: (b, 0, 0)),
            pl.BlockSpec((1, 1, N), lambda b: (b, 0, 0)),
        ],
        out_specs=pl.BlockSpec((1, NC, H), lambda b: (b, 0, 0)),
        out_shape=jax.ShapeDtypeStruct((B, NC, H), f32),
    )(fine_features, atom_to_coarse.reshape(B, 1, N))

    gw, gsum = pl.pallas_call(
        _glob_body,
        grid=(B,),
        in_specs=[
            pl.BlockSpec((1, N, H), lambda b: (b, 0, 0)),
            pl.BlockSpec((H, H), lambda b: (0, 0)),
            pl.BlockSpec((1, H), lambda b: (0, 0)),
        ],
        out_specs=[
            pl.BlockSpec((1, N, H), lambda b: (b, 0, 0)),
            pl.BlockSpec((1, 1, H), lambda b: (b, 0, 0)),
        ],
        out_shape=[
            jax.ShapeDtypeStruct((B, N, H), f32),
            jax.ShapeDtypeStruct((B, 1, H), f32),
        ],
    )(global_features, wgi2.astype(jnp.bfloat16), b_gi.reshape(1, H))

    vecs = jnp.stack([b_f2c, g1, be1, b_c2f, g2, be2, b_gate, b_gi, g3, be3])
    ws = jnp.stack([W_f2c, W_c2f, wg1, wg2, wgi1, wgi2]).astype(jnp.bfloat16)
    coarse_out, tab = pl.pallas_call(
        _coarse_body,
        grid=(B,),
        in_specs=[
            pl.BlockSpec((1, NC, H), lambda b: (b, 0, 0)),
            pl.BlockSpec((1, NC, H), lambda b: (b, 0, 0)),
            pl.BlockSpec((1, 1, H), lambda b: (b, 0, 0)),
            pl.BlockSpec((6, H, H), lambda b: (0, 0, 0)),
            pl.BlockSpec((10, H), lambda b: (0, 0)),
        ],
        out_specs=[
            pl.BlockSpec((1, NC, H), lambda b: (b, 0, 0)),
            pl.BlockSpec((1, NC, 2 * H), lambda b: (b, 0, 0)),
        ],
        out_shape=[
            jax.ShapeDtypeStruct((B, NC, H), f32),
            jax.ShapeDtypeStruct((B, NC, 2 * H), f32),
        ],
    )(seg_mean, coarse_features, gsum, ws, vecs)

    a2c_g = atom_to_coarse.reshape(B * N // _G_CH, _G_CH)
    ctx = _gather_call(tab.reshape(B * NC, 2 * H), a2c_g)
    ctx = ctx.reshape(B * N, 2 * H)

    RB = 1024
    vec3 = jnp.stack([b_gate, g3, be3])
    fine_out = pl.pallas_call(
        _fine_body,
        grid=(B * N // RB,),
        in_specs=[
            pl.BlockSpec((RB, H), lambda i: (i, 0)),
            pl.BlockSpec((RB, 2 * H), lambda i: (i, 0)),
            pl.BlockSpec((RB, H), lambda i: (i, 0)),
            pl.BlockSpec((H, H), lambda i: (0, 0)),
            pl.BlockSpec((H, H), lambda i: (0, 0)),
            pl.BlockSpec((3, H), lambda i: (0, 0)),
        ],
        out_specs=pl.BlockSpec((RB, H), lambda i: (i, 0)),
        out_shape=jax.ShapeDtypeStruct((B * N, H), f32),
    )(fine_features.reshape(B * N, H), ctx, gw.reshape(B * N, H),
      wg1.astype(jnp.bfloat16), wgi1.astype(jnp.bfloat16), vec3)

    return fine_out.reshape(B, N, H), coarse_out


# speedup vs baseline: 3.8414x; 1.0895x over previous
import functools

import jax
import jax.numpy as jnp
from jax import lax
from jax.experimental import pallas as pl
from jax.experimental.pallas import tpu as pltpu
from jax.experimental.pallas import tpu_sc as plsc

_B, _N, _NC, _H = 8, 4096, 512, 512
_F32 = jnp.float32

_NCORES = 2
_NSUB = 16


@functools.lru_cache(maxsize=None)
def _make_gather(W):
    NW = _NCORES * _NSUB
    APT = _B * _N // NW
    CH = 32
    NCH = APT // CH
    WPB = NW // _B
    mesh = plsc.VectorSubcoreMesh(core_axis_name="c", subcore_axis_name="s",
                                  num_cores=_NCORES, num_subcores=_NSUB)

    @functools.partial(
        pl.kernel,
        out_type=jax.ShapeDtypeStruct((_B * _N // CH, CH, W), _F32),
        mesh=mesh,
    scratch_types=[
        pltpu.VMEM((CH,), jnp.int32),
        pltpu.VMEM((CH,), jnp.int32),
        pltpu.VMEM((CH, W), _F32),
        pltpu.VMEM((CH, W), _F32),
        pltpu.SemaphoreType.DMA,
        pltpu.SemaphoreType.DMA,
        pltpu.SemaphoreType.DMA,
        pltpu.SemaphoreType.DMA,
    ],
    )
    def gather(tab_hbm, a2c_hbm, out_hbm, idx_a, idx_b, rows_a, rows_b,
               sem_a, sem_b, sem_oa, sem_ob):
        c = lax.axis_index("c")
        s = lax.axis_index("s")
        wid = s * _NCORES + c
        base = jnp.full((16,), wid // WPB * _NC, jnp.int32)

        def load_idx(ch, idx_v):
            blk = wid * NCH + ch
            pltpu.sync_copy(a2c_hbm.at[blk], idx_v)
            for j in range(CH // 16):
                sl = pl.ds(j * 16, 16)
                idx_v[sl] = idx_v[sl] + base

        load_idx(0, idx_a)
        pltpu.async_copy(tab_hbm.at[idx_a], rows_a, sem_a)

        def body(gg, _):
            g0 = gg * 2
            pltpu.make_async_copy(tab_hbm.at[idx_a], rows_a, sem_a).wait()
            pltpu.async_copy(rows_a, out_hbm.at[wid * NCH + g0], sem_oa)
            @pl.when(gg > 0)
            def _():
                pltpu.make_async_copy(rows_b, out_hbm.at[0], sem_ob).wait()
            load_idx(g0 + 1, idx_b)
            pltpu.async_copy(tab_hbm.at[idx_b], rows_b, sem_b)
            @pl.when(gg < NCH // 2 - 1)
            def _():
                pltpu.make_async_copy(rows_a, out_hbm.at[0], sem_oa).wait()
                load_idx(g0 + 2, idx_a)
                pltpu.async_copy(tab_hbm.at[idx_a], rows_a, sem_a)
            pltpu.make_async_copy(tab_hbm.at[idx_b], rows_b, sem_b).wait()
            pltpu.async_copy(rows_b, out_hbm.at[wid * NCH + g0 + 1], sem_ob)
            return 0
        lax.fori_loop(0, NCH // 2, body, 0)
        pltpu.make_async_copy(rows_a, out_hbm.at[0], sem_oa).wait()
        pltpu.make_async_copy(rows_b, out_hbm.at[0], sem_ob).wait()

    return gather, CH


_G_CH = 32


def _gather_call(tab, a2c_g):
    return _make_gather(2 * _H)[0](tab, a2c_g)


def _ln(x, g, b):
    m = jnp.mean(x, axis=-1, keepdims=True)
    v = jnp.mean((x - m) ** 2, axis=-1, keepdims=True)
    return (x - m) / jnp.sqrt(v + 1e-5) * g + b


def _sigmoid(x):
    return 1.0 / (1.0 + jnp.exp(-x))


_BF16 = jnp.bfloat16


def _bdot(x, w):
    return jnp.dot(x.astype(_BF16), w, preferred_element_type=_F32)


def _glob_body(g_ref, w_ref, b_ref, gw_ref, gs_ref):
    g = g_ref[0]
    gw_ref[0] = _bdot(g, w_ref[...]) + b_ref[...]
    gs_ref[0] = jnp.sum(g, axis=0, keepdims=True)


def _seg_mean_body(f_ref, a_ref, out_ref):
    idx = a_ref[0, 0]
    iota = lax.broadcasted_iota(jnp.int32, (_N, _NC), 1)
    hot = (idx[:, None] == iota)
    cnt = jnp.sum(hot.astype(_F32), axis=0)
    rec = 1.0 / jnp.maximum(cnt, 1.0)
    sums = lax.dot_general(hot.astype(jnp.bfloat16),
                           f_ref[0].astype(jnp.bfloat16),
                           (((0,), (0,)), ((), ())),
                           preferred_element_type=_F32)
    out_ref[0] = sums * rec[:, None]


def _coarse_body(m_ref, co_ref, gs_ref, ws_ref, vec_ref, out_ref, tab_ref):
    b_f2c, g1, be1 = vec_ref[0:1], vec_ref[1:2], vec_ref[2:3]
    b_c2f, g2, be2 = vec_ref[3:4], vec_ref[4:5], vec_ref[5:6]
    b_gate, b_gi = vec_ref[6:7], vec_ref[7:8]
    g3, be3 = vec_ref[8:9], vec_ref[9:10]
    w_f2c, w_c2f = ws_ref[0], ws_ref[1]
    wg1, wg2, wgi1, wgi2 = ws_ref[2], ws_ref[3], ws_ref[4], ws_ref[5]

    coarse = co_ref[0]
    cff = jnp.maximum(_ln(_bdot(m_ref[0], w_f2c) + b_f2c, g1, be1), 0.0)
    c2 = jnp.maximum(_ln(_bdot(coarse, w_c2f) + b_c2f, g2, be2), 0.0)
    c3 = _bdot(c2, wg2)
    cg = _sigmoid(_bdot(coarse, wg1) + _bdot(cff, wg2) + b_gate)
    cu = cg * coarse + (1.0 - cg) * cff
    gc = gs_ref[0] * (1.0 / _N)
    cwg = jnp.maximum(
        _ln(_bdot(cu, wgi1) + _bdot(gc, wgi2) + b_gi, g3, be3), 0.0)
    out_ref[0] = cu + 0.1 * cwg
    tab_ref[0] = jnp.concatenate([c2, c3], axis=-1)


def _fine_body(f_ref, r_ref, gw_ref, wg1_ref, wgi1_ref, vec_ref, out_ref):
    b_gate, g3, be3 = vec_ref[0:1], vec_ref[1:2], vec_ref[2:3]
    f = f_ref[...]
    c2g = r_ref[:, :_H]
    g3row = r_ref[:, _H:]
    fg = _sigmoid(_bdot(f, wg1_ref[...]) + g3row + b_gate)
    fu = fg * f + (1.0 - fg) * c2g
    fwg = jnp.maximum(
        _ln(_bdot(fu, wgi1_ref[...]) + gw_ref[...], g3, be3), 0.0)
    out_ref[...] = fu + 0.1 * fwg


def kernel(fine_features, coarse_features, atom_to_coarse, global_features,
           W_f2c, b_f2c, g1, be1, W_c2f, b_c2f, g2, be2,
           W_gate, b_gate, W_gi, b_gi, g3, be3):
    B, N, NC, H = _B, _N, _NC, _H
    f32 = _F32

    wg1, wg2 = W_gate[:H], W_gate[H:]
    wgi1, wgi2 = W_gi[:H], W_gi[H:]

    seg_mean = pl.pallas_call(
        _seg_mean_body,
        grid=(B,),
        in_specs=[
            pl.BlockSpec((1, N, H), lambda b: (b, 0, 0)),
            pl.BlockSpec((1, 1, N), lambda b: (b, 0, 0)),
        ],
        out_specs=pl.BlockSpec((1, NC, H), lambda b: (b, 0, 0)),
        out_shape=jax.ShapeDtypeStruct((B, NC, H), f32),
    )(fine_features, atom_to_coarse.reshape(B, 1, N))

    gw, gsum = pl.pallas_call(
        _glob_body,
        grid=(B,),
        in_specs=[
            pl.BlockSpec((1, N, H), lambda b: (b, 0, 0)),
            pl.BlockSpec((H, H), lambda b: (0, 0)),
            pl.BlockSpec((1, H), lambda b: (0, 0)),
        ],
        out_specs=[
            pl.BlockSpec((1, N, H), lambda b: (b, 0, 0)),
            pl.BlockSpec((1, 1, H), lambda b: (b, 0, 0)),
        ],
        out_shape=[
            jax.ShapeDtypeStruct((B, N, H), f32),
            jax.ShapeDtypeStruct((B, 1, H), f32),
        ],
    )(global_features, wgi2.astype(jnp.bfloat16), b_gi.reshape(1, H))

    vecs = jnp.stack([b_f2c, g1, be1, b_c2f, g2, be2, b_gate, b_gi, g3, be3])
    ws = jnp.stack([W_f2c, W_c2f, wg1, wg2, wgi1, wgi2]).astype(jnp.bfloat16)
    coarse_out, tab = pl.pallas_call(
        _coarse_body,
        grid=(B,),
        in_specs=[
            pl.BlockSpec((1, NC, H), lambda b: (b, 0, 0)),
            pl.BlockSpec((1, NC, H), lambda b: (b, 0, 0)),
            pl.BlockSpec((1, 1, H), lambda b: (b, 0, 0)),
            pl.BlockSpec((6, H, H), lambda b: (0, 0, 0)),
            pl.BlockSpec((10, H), lambda b: (0, 0)),
        ],
        out_specs=[
            pl.BlockSpec((1, NC, H), lambda b: (b, 0, 0)),
            pl.BlockSpec((1, NC, 2 * H), lambda b: (b, 0, 0)),
        ],
        out_shape=[
            jax.ShapeDtypeStruct((B, NC, H), f32),
            jax.ShapeDtypeStruct((B, NC, 2 * H), f32),
        ],
    )(seg_mean, coarse_features, gsum, ws, vecs)

    a2c_g = atom_to_coarse.reshape(B * N // _G_CH, _G_CH)
    ctx = _gather_call(tab.reshape(B * NC, 2 * H), a2c_g)
    ctx = ctx.reshape(B * N, 2 * H)

    RB = 1024
    vec3 = jnp.stack([b_gate, g3, be3])
    fine_out = pl.pallas_call(
        _fine_body,
        grid=(B * N // RB,),
        in_specs=[
            pl.BlockSpec((RB, H), lambda i: (i, 0)),
            pl.BlockSpec((RB, 2 * H), lambda i: (i, 0)),
            pl.BlockSpec((RB, H), lambda i: (i, 0)),
            pl.BlockSpec((H, H), lambda i: (0, 0)),
            pl.BlockSpec((H, H), lambda i: (0, 0)),
            pl.BlockSpec((3, H), lambda i: (0, 0)),
        ],
        out_specs=pl.BlockSpec((RB, H), lambda i: (i, 0)),
        out_shape=jax.ShapeDtypeStruct((B * N, H), f32),
    )(fine_features.reshape(B * N, H), ctx, gw.reshape(B * N, H),
      wg1.astype(jnp.bfloat16), wgi1.astype(jnp.bfloat16), vec3)

    return fine_out.reshape(B, N, H), coarse_out


# speedup vs baseline: 4.4672x; 1.1629x over previous
import functools

import jax
import jax.numpy as jnp
from jax import lax
from jax.experimental import pallas as pl
from jax.experimental.pallas import tpu as pltpu
from jax.experimental.pallas import tpu_sc as plsc

_B, _N, _NC, _H = 8, 4096, 512, 512
_F32 = jnp.float32

_NCORES = 2
_NSUB = 16


@functools.lru_cache(maxsize=None)
def _make_gather(W):
    NW = _NCORES * _NSUB
    APT = _B * _N // NW
    CH = 32
    NCH = APT // CH
    WPB = NW // _B
    mesh = plsc.VectorSubcoreMesh(core_axis_name="c", subcore_axis_name="s",
                                  num_cores=_NCORES, num_subcores=_NSUB)

    @functools.partial(
        pl.kernel,
        out_type=jax.ShapeDtypeStruct((_B * _N // CH, CH, W), jnp.int32),
        mesh=mesh,
    scratch_types=[
        pltpu.VMEM((CH,), jnp.int32),
        pltpu.VMEM((CH,), jnp.int32),
        pltpu.VMEM((CH, W), jnp.int32),
        pltpu.VMEM((CH, W), jnp.int32),
        pltpu.SemaphoreType.DMA,
        pltpu.SemaphoreType.DMA,
        pltpu.SemaphoreType.DMA,
        pltpu.SemaphoreType.DMA,
    ],
    )
    def gather(tab_hbm, a2c_hbm, out_hbm, idx_a, idx_b, rows_a, rows_b,
               sem_a, sem_b, sem_oa, sem_ob):
        c = lax.axis_index("c")
        s = lax.axis_index("s")
        wid = s * _NCORES + c
        base = jnp.full((16,), wid // WPB * _NC, jnp.int32)

        def load_idx(ch, idx_v):
            blk = wid * NCH + ch
            pltpu.sync_copy(a2c_hbm.at[blk], idx_v)
            for j in range(CH // 16):
                sl = pl.ds(j * 16, 16)
                idx_v[sl] = idx_v[sl] + base

        load_idx(0, idx_a)
        pltpu.async_copy(tab_hbm.at[idx_a], rows_a, sem_a)

        def body(gg, _):
            g0 = gg * 2
            pltpu.make_async_copy(tab_hbm.at[idx_a], rows_a, sem_a).wait()
            pltpu.async_copy(rows_a, out_hbm.at[wid * NCH + g0], sem_oa)
            @pl.when(gg > 0)
            def _():
                pltpu.make_async_copy(rows_b, out_hbm.at[0], sem_ob).wait()
            load_idx(g0 + 1, idx_b)
            pltpu.async_copy(tab_hbm.at[idx_b], rows_b, sem_b)
            @pl.when(gg < NCH // 2 - 1)
            def _():
                pltpu.make_async_copy(rows_a, out_hbm.at[0], sem_oa).wait()
                load_idx(g0 + 2, idx_a)
                pltpu.async_copy(tab_hbm.at[idx_a], rows_a, sem_a)
            pltpu.make_async_copy(tab_hbm.at[idx_b], rows_b, sem_b).wait()
            pltpu.async_copy(rows_b, out_hbm.at[wid * NCH + g0 + 1], sem_ob)
            return 0
        lax.fori_loop(0, NCH // 2, body, 0)
        pltpu.make_async_copy(rows_a, out_hbm.at[0], sem_oa).wait()
        pltpu.make_async_copy(rows_b, out_hbm.at[0], sem_ob).wait()

    return gather, CH


_G_CH = 32


def _gather_call(tab, a2c_g):
    return _make_gather(_H)[0](tab, a2c_g)


def _ln(x, g, b):
    m = jnp.mean(x, axis=-1, keepdims=True)
    v = jnp.mean((x - m) ** 2, axis=-1, keepdims=True)
    return (x - m) / jnp.sqrt(v + 1e-5) * g + b


def _sigmoid(x):
    return 1.0 / (1.0 + jnp.exp(-x))


_BF16 = jnp.bfloat16


def _bdot(x, w):
    return jnp.dot(x.astype(_BF16), w, preferred_element_type=_F32)


def _glob_body(g_ref, w_ref, b_ref, gw_ref, gs_ref):
    g = g_ref[0]
    gw_ref[0] = _bdot(g, w_ref[...]) + b_ref[...]
    gs_ref[0] = jnp.sum(g, axis=0, keepdims=True)


def _seg_mean_body(f_ref, a_ref, out_ref):
    idx = a_ref[0, 0]
    iota = lax.broadcasted_iota(jnp.int32, (_N, _NC), 1)
    hot = (idx[:, None] == iota)
    cnt = jnp.sum(hot.astype(_F32), axis=0)
    rec = 1.0 / jnp.maximum(cnt, 1.0)
    sums = lax.dot_general(hot.astype(jnp.bfloat16),
                           f_ref[0].astype(jnp.bfloat16),
                           (((0,), (0,)), ((), ())),
                           preferred_element_type=_F32)
    out_ref[0] = sums * rec[:, None]


def _coarse_body(m_ref, co_ref, gs_ref, ws_ref, vec_ref, out_ref, tab_ref):
    b_f2c, g1, be1 = vec_ref[0:1], vec_ref[1:2], vec_ref[2:3]
    b_c2f, g2, be2 = vec_ref[3:4], vec_ref[4:5], vec_ref[5:6]
    b_gate, b_gi = vec_ref[6:7], vec_ref[7:8]
    g3, be3 = vec_ref[8:9], vec_ref[9:10]
    w_f2c, w_c2f = ws_ref[0], ws_ref[1]
    wg1, wg2, wgi1, wgi2 = ws_ref[2], ws_ref[3], ws_ref[4], ws_ref[5]

    coarse = co_ref[0]
    cff = jnp.maximum(_ln(_bdot(m_ref[0], w_f2c) + b_f2c, g1, be1), 0.0)
    c2 = jnp.maximum(_ln(_bdot(coarse, w_c2f) + b_c2f, g2, be2), 0.0)
    c3 = _bdot(c2, wg2)
    cg = _sigmoid(_bdot(coarse, wg1) + _bdot(cff, wg2) + b_gate)
    cu = cg * coarse + (1.0 - cg) * cff
    gc = gs_ref[0] * (1.0 / _N)
    cwg = jnp.maximum(
        _ln(_bdot(cu, wgi1) + _bdot(gc, wgi2) + b_gi, g3, be3), 0.0)
    out_ref[0] = cu + 0.1 * cwg
    lob = lax.bitcast_convert_type(c2, jnp.int32)
    lob = lob + 0x7FFF + ((lob >> 16) & 1)
    hib = lax.bitcast_convert_type(c3, jnp.int32)
    hib = hib + 0x7FFF + ((hib >> 16) & 1)
    tab_ref[0] = (hib & jnp.int32(-65536)) | lax.shift_right_logical(lob, 16)


def _fine_body(f_ref, r_ref, gw_ref, wg1_ref, wgi1_ref, vec_ref, out_ref):
    b_gate, g3, be3 = vec_ref[0:1], vec_ref[1:2], vec_ref[2:3]
    f = f_ref[...]
    w = r_ref[...]
    c2g = lax.bitcast_convert_type(lax.shift_left(w, 16), _F32)
    g3row = lax.bitcast_convert_type(w & jnp.int32(-65536), _F32)
    fg = _sigmoid(_bdot(f, wg1_ref[...]) + g3row + b_gate)
    fu = fg * f + (1.0 - fg) * c2g
    fwg = jnp.maximum(
        _ln(_bdot(fu, wgi1_ref[...]) + gw_ref[...], g3, be3), 0.0)
    out_ref[...] = fu + 0.1 * fwg


def kernel(fine_features, coarse_features, atom_to_coarse, global_features,
           W_f2c, b_f2c, g1, be1, W_c2f, b_c2f, g2, be2,
           W_gate, b_gate, W_gi, b_gi, g3, be3):
    B, N, NC, H = _B, _N, _NC, _H
    f32 = _F32

    wg1, wg2 = W_gate[:H], W_gate[H:]
    wgi1, wgi2 = W_gi[:H], W_gi[H:]

    seg_mean = pl.pallas_call(
        _seg_mean_body,
        grid=(B,),
        in_specs=[
            pl.BlockSpec((1, N, H), lambda b: (b, 0, 0)),
            pl.BlockSpec((1, 1, N), lambda b: (b, 0, 0)),
        ],
        out_specs=pl.BlockSpec((1, NC, H), lambda b: (b, 0, 0)),
        out_shape=jax.ShapeDtypeStruct((B, NC, H), f32),
    )(fine_features, atom_to_coarse.reshape(B, 1, N))

    gw, gsum = pl.pallas_call(
        _glob_body,
        grid=(B,),
        in_specs=[
            pl.BlockSpec((1, N, H), lambda b: (b, 0, 0)),
            pl.BlockSpec((H, H), lambda b: (0, 0)),
            pl.BlockSpec((1, H), lambda b: (0, 0)),
        ],
        out_specs=[
            pl.BlockSpec((1, N, H), lambda b: (b, 0, 0)),
            pl.BlockSpec((1, 1, H), lambda b: (b, 0, 0)),
        ],
        out_shape=[
            jax.ShapeDtypeStruct((B, N, H), f32),
            jax.ShapeDtypeStruct((B, 1, H), f32),
        ],
    )(global_features, wgi2.astype(jnp.bfloat16), b_gi.reshape(1, H))

    vecs = jnp.stack([b_f2c, g1, be1, b_c2f, g2, be2, b_gate, b_gi, g3, be3])
    ws = jnp.stack([W_f2c, W_c2f, wg1, wg2, wgi1, wgi2]).astype(jnp.bfloat16)
    coarse_out, tab = pl.pallas_call(
        _coarse_body,
        grid=(B,),
        in_specs=[
            pl.BlockSpec((1, NC, H), lambda b: (b, 0, 0)),
            pl.BlockSpec((1, NC, H), lambda b: (b, 0, 0)),
            pl.BlockSpec((1, 1, H), lambda b: (b, 0, 0)),
            pl.BlockSpec((6, H, H), lambda b: (0, 0, 0)),
            pl.BlockSpec((10, H), lambda b: (0, 0)),
        ],
        out_specs=[
            pl.BlockSpec((1, NC, H), lambda b: (b, 0, 0)),
            pl.BlockSpec((1, NC, H), lambda b: (b, 0, 0)),
        ],
        out_shape=[
            jax.ShapeDtypeStruct((B, NC, H), f32),
            jax.ShapeDtypeStruct((B, NC, H), jnp.int32),
        ],
    )(seg_mean, coarse_features, gsum, ws, vecs)

    a2c_g = atom_to_coarse.reshape(B * N // _G_CH, _G_CH)
    ctx = _gather_call(tab.reshape(B * NC, H), a2c_g)
    ctx = ctx.reshape(B * N, H)

    RB = 1024
    vec3 = jnp.stack([b_gate, g3, be3])
    fine_out = pl.pallas_call(
        _fine_body,
        grid=(B * N // RB,),
        in_specs=[
            pl.BlockSpec((RB, H), lambda i: (i, 0)),
            pl.BlockSpec((RB, H), lambda i: (i, 0)),
            pl.BlockSpec((RB, H), lambda i: (i, 0)),
            pl.BlockSpec((H, H), lambda i: (0, 0)),
            pl.BlockSpec((H, H), lambda i: (0, 0)),
            pl.BlockSpec((3, H), lambda i: (0, 0)),
        ],
        out_specs=pl.BlockSpec((RB, H), lambda i: (i, 0)),
        out_shape=jax.ShapeDtypeStruct((B * N, H), f32),
    )(fine_features.reshape(B * N, H), ctx, gw.reshape(B * N, H),
      wg1.astype(jnp.bfloat16), wgi1.astype(jnp.bfloat16), vec3)

    return fine_out.reshape(B, N, H), coarse_out


# speedup vs baseline: 4.8856x; 1.0937x over previous
import functools

import jax
import jax.numpy as jnp
from jax import lax
from jax.experimental import pallas as pl
from jax.experimental.pallas import tpu as pltpu
from jax.experimental.pallas import tpu_sc as plsc

_B, _N, _NC, _H = 8, 4096, 512, 512
_F32 = jnp.float32

_NCORES = 2
_NSUB = 16


@functools.lru_cache(maxsize=None)
def _make_gather(W):
    NW = _NCORES * _NSUB
    APT = _B * _N // NW
    CH = 32
    NCH = APT // CH
    WPB = NW // _B
    mesh = plsc.VectorSubcoreMesh(core_axis_name="c", subcore_axis_name="s",
                                  num_cores=_NCORES, num_subcores=_NSUB)

    @functools.partial(
        pl.kernel,
        out_type=jax.ShapeDtypeStruct((_B * _N // CH, CH, W), jnp.int32),
        mesh=mesh,
    scratch_types=[
        pltpu.VMEM((CH,), jnp.int32),
        pltpu.VMEM((CH,), jnp.int32),
        pltpu.VMEM((CH, W), jnp.int32),
        pltpu.VMEM((CH, W), jnp.int32),
        pltpu.SemaphoreType.DMA,
        pltpu.SemaphoreType.DMA,
        pltpu.SemaphoreType.DMA,
        pltpu.SemaphoreType.DMA,
    ],
    )
    def gather(tab_hbm, a2c_hbm, out_hbm, idx_a, idx_b, rows_a, rows_b,
               sem_a, sem_b, sem_oa, sem_ob):
        c = lax.axis_index("c")
        s = lax.axis_index("s")
        wid = s * _NCORES + c
        base = jnp.full((16,), wid // WPB * _NC, jnp.int32)

        def load_idx(ch, idx_v):
            blk = wid * NCH + ch
            pltpu.sync_copy(a2c_hbm.at[blk], idx_v)
            for j in range(CH // 16):
                sl = pl.ds(j * 16, 16)
                idx_v[sl] = idx_v[sl] + base

        load_idx(0, idx_a)
        pltpu.async_copy(tab_hbm.at[idx_a], rows_a, sem_a)

        def body(gg, _):
            g0 = gg * 2
            pltpu.make_async_copy(tab_hbm.at[idx_a], rows_a, sem_a).wait()
            pltpu.async_copy(rows_a, out_hbm.at[wid * NCH + g0], sem_oa)
            @pl.when(gg > 0)
            def _():
                pltpu.make_async_copy(rows_b, out_hbm.at[0], sem_ob).wait()
            load_idx(g0 + 1, idx_b)
            pltpu.async_copy(tab_hbm.at[idx_b], rows_b, sem_b)
            @pl.when(gg < NCH // 2 - 1)
            def _():
                pltpu.make_async_copy(rows_a, out_hbm.at[0], sem_oa).wait()
                load_idx(g0 + 2, idx_a)
                pltpu.async_copy(tab_hbm.at[idx_a], rows_a, sem_a)
            pltpu.make_async_copy(tab_hbm.at[idx_b], rows_b, sem_b).wait()
            pltpu.async_copy(rows_b, out_hbm.at[wid * NCH + g0 + 1], sem_ob)
            return 0
        lax.fori_loop(0, NCH // 2, body, 0)
        pltpu.make_async_copy(rows_a, out_hbm.at[0], sem_oa).wait()
        pltpu.make_async_copy(rows_b, out_hbm.at[0], sem_ob).wait()

    return gather, CH


_G_CH = 32


def _gather_call(tab, a2c_g):
    return _make_gather(_H)[0](tab, a2c_g)


def _ln(x, g, b):
    m = jnp.mean(x, axis=-1, keepdims=True)
    v = jnp.mean((x - m) ** 2, axis=-1, keepdims=True)
    return (x - m) / jnp.sqrt(v + 1e-5) * g + b


def _sigmoid(x):
    return 1.0 / (1.0 + jnp.exp(-x))


_BF16 = jnp.bfloat16


def _bdot(x, w):
    return jnp.dot(x.astype(_BF16), w, preferred_element_type=_F32)


def _seg_mean_body(f_ref, a_ref, g_ref, out_ref, gs_ref):
    idx = a_ref[0, 0]
    iota = lax.broadcasted_iota(jnp.int32, (_N, _NC), 1)
    hot = (idx[:, None] == iota)
    cnt = jnp.sum(hot.astype(_F32), axis=0)
    rec = 1.0 / jnp.maximum(cnt, 1.0)
    sums = lax.dot_general(hot.astype(jnp.bfloat16),
                           f_ref[0].astype(jnp.bfloat16),
                           (((0,), (0,)), ((), ())),
                           preferred_element_type=_F32)
    out_ref[0] = sums * rec[:, None]
    gs_ref[0] = jnp.sum(g_ref[0], axis=0, keepdims=True)


def _coarse_body(m_ref, co_ref, gs_ref, ws_ref, vec_ref, out_ref, tab_ref):
    b_f2c, g1, be1 = vec_ref[0:1], vec_ref[1:2], vec_ref[2:3]
    b_c2f, g2, be2 = vec_ref[3:4], vec_ref[4:5], vec_ref[5:6]
    b_gate, b_gi = vec_ref[6:7], vec_ref[7:8]
    g3, be3 = vec_ref[8:9], vec_ref[9:10]
    w_f2c, w_c2f = ws_ref[0], ws_ref[1]
    wg1, wg2, wgi1, wgi2 = ws_ref[2], ws_ref[3], ws_ref[4], ws_ref[5]

    coarse = co_ref[0]
    cff = jnp.maximum(_ln(_bdot(m_ref[0], w_f2c) + b_f2c, g1, be1), 0.0)
    c2 = jnp.maximum(_ln(_bdot(coarse, w_c2f) + b_c2f, g2, be2), 0.0)
    c3 = _bdot(c2, wg2)
    cg = _sigmoid(_bdot(coarse, wg1) + _bdot(cff, wg2) + b_gate)
    cu = cg * coarse + (1.0 - cg) * cff
    gc = gs_ref[0] * (1.0 / _N)
    cwg = jnp.maximum(
        _ln(_bdot(cu, wgi1) + _bdot(gc, wgi2) + b_gi, g3, be3), 0.0)
    out_ref[0] = cu + 0.1 * cwg
    lob = lax.bitcast_convert_type(c2, jnp.int32)
    lob = lob + 0x7FFF + ((lob >> 16) & 1)
    hib = lax.bitcast_convert_type(c3, jnp.int32)
    hib = hib + 0x7FFF + ((hib >> 16) & 1)
    tab_ref[0] = (hib & jnp.int32(-65536)) | lax.shift_right_logical(lob, 16)


def _fine_body(f_ref, r_ref, g_ref, wg1_ref, wgi1_ref, wgi2_ref, vec_ref,
               out_ref):
    b_gate, b_gi = vec_ref[0:1], vec_ref[1:2]
    g3, be3 = vec_ref[2:3], vec_ref[3:4]
    f = f_ref[...]
    w = r_ref[...]
    c2g = lax.bitcast_convert_type(lax.shift_left(w, 16), _F32)
    g3row = lax.bitcast_convert_type(w & jnp.int32(-65536), _F32)
    fg = _sigmoid(_bdot(f, wg1_ref[...]) + g3row + b_gate)
    fu = fg * f + (1.0 - fg) * c2g
    fwg = jnp.maximum(
        _ln(_bdot(fu, wgi1_ref[...]) + _bdot(g_ref[...], wgi2_ref[...])
            + b_gi, g3, be3), 0.0)
    out_ref[...] = fu + 0.1 * fwg


def kernel(fine_features, coarse_features, atom_to_coarse, global_features,
           W_f2c, b_f2c, g1, be1, W_c2f, b_c2f, g2, be2,
           W_gate, b_gate, W_gi, b_gi, g3, be3):
    B, N, NC, H = _B, _N, _NC, _H
    f32 = _F32

    wg1, wg2 = W_gate[:H], W_gate[H:]
    wgi1, wgi2 = W_gi[:H], W_gi[H:]

    seg_mean, gsum = pl.pallas_call(
        _seg_mean_body,
        grid=(B,),
        in_specs=[
            pl.BlockSpec((1, N, H), lambda b: (b, 0, 0)),
            pl.BlockSpec((1, 1, N), lambda b: (b, 0, 0)),
            pl.BlockSpec((1, N, H), lambda b: (b, 0, 0)),
        ],
        out_specs=[
            pl.BlockSpec((1, NC, H), lambda b: (b, 0, 0)),
            pl.BlockSpec((1, 1, H), lambda b: (b, 0, 0)),
        ],
        out_shape=[
            jax.ShapeDtypeStruct((B, NC, H), f32),
            jax.ShapeDtypeStruct((B, 1, H), f32),
        ],
    )(fine_features, atom_to_coarse.reshape(B, 1, N), global_features)

    vecs = jnp.stack([b_f2c, g1, be1, b_c2f, g2, be2, b_gate, b_gi, g3, be3])
    ws = jnp.stack([W_f2c, W_c2f, wg1, wg2, wgi1, wgi2]).astype(jnp.bfloat16)
    coarse_out, tab = pl.pallas_call(
        _coarse_body,
        grid=(B,),
        in_specs=[
            pl.BlockSpec((1, NC, H), lambda b: (b, 0, 0)),
            pl.BlockSpec((1, NC, H), lambda b: (b, 0, 0)),
            pl.BlockSpec((1, 1, H), lambda b: (b, 0, 0)),
            pl.BlockSpec((6, H, H), lambda b: (0, 0, 0)),
            pl.BlockSpec((10, H), lambda b: (0, 0)),
        ],
        out_specs=[
            pl.BlockSpec((1, NC, H), lambda b: (b, 0, 0)),
            pl.BlockSpec((1, NC, H), lambda b: (b, 0, 0)),
        ],
        out_shape=[
            jax.ShapeDtypeStruct((B, NC, H), f32),
            jax.ShapeDtypeStruct((B, NC, H), jnp.int32),
        ],
    )(seg_mean, coarse_features, gsum, ws, vecs)

    a2c_g = atom_to_coarse.reshape(B * N // _G_CH, _G_CH)
    ctx = _gather_call(tab.reshape(B * NC, H), a2c_g)
    ctx = ctx.reshape(B * N, H)

    RB = 1024
    vec4 = jnp.stack([b_gate, b_gi, g3, be3])
    fine_out = pl.pallas_call(
        _fine_body,
        grid=(B * N // RB,),
        in_specs=[
            pl.BlockSpec((RB, H), lambda i: (i, 0)),
            pl.BlockSpec((RB, H), lambda i: (i, 0)),
            pl.BlockSpec((RB, H), lambda i: (i, 0)),
            pl.BlockSpec((H, H), lambda i: (0, 0)),
            pl.BlockSpec((H, H), lambda i: (0, 0)),
            pl.BlockSpec((H, H), lambda i: (0, 0)),
            pl.BlockSpec((4, H), lambda i: (0, 0)),
        ],
        out_specs=pl.BlockSpec((RB, H), lambda i: (i, 0)),
        out_shape=jax.ShapeDtypeStruct((B * N, H), f32),
    )(fine_features.reshape(B * N, H), ctx,
      global_features.reshape(B * N, H),
      wg1.astype(jnp.bfloat16), wgi1.astype(jnp.bfloat16),
      wgi2.astype(jnp.bfloat16), vec4)

    return fine_out.reshape(B, N, H), coarse_out


# speedup vs baseline: 5.2045x; 1.0653x over previous
import functools

import jax
import jax.numpy as jnp
from jax import lax
from jax.experimental import pallas as pl
from jax.experimental.pallas import tpu as pltpu
from jax.experimental.pallas import tpu_sc as plsc

_B, _N, _NC, _H = 8, 4096, 512, 512
_F32 = jnp.float32

_NCORES = 2
_NSUB = 16


@functools.lru_cache(maxsize=None)
def _make_gather(W):
    NW = _NCORES * _NSUB
    APT = _B * _N // NW
    CH = 32
    NCH = APT // CH
    WPB = NW // _B
    mesh = plsc.VectorSubcoreMesh(core_axis_name="c", subcore_axis_name="s",
                                  num_cores=_NCORES, num_subcores=_NSUB)

    @functools.partial(
        pl.kernel,
        out_type=jax.ShapeDtypeStruct((_B * _N // CH, CH, W), jnp.int32),
        mesh=mesh,
    scratch_types=[
        pltpu.VMEM((CH,), jnp.int32),
        pltpu.VMEM((CH,), jnp.int32),
        pltpu.VMEM((CH, W), jnp.int32),
        pltpu.VMEM((CH, W), jnp.int32),
        pltpu.SemaphoreType.DMA,
        pltpu.SemaphoreType.DMA,
        pltpu.SemaphoreType.DMA,
        pltpu.SemaphoreType.DMA,
    ],
    )
    def gather(tab_hbm, a2c_hbm, out_hbm, idx_a, idx_b, rows_a, rows_b,
               sem_a, sem_b, sem_oa, sem_ob):
        c = lax.axis_index("c")
        s = lax.axis_index("s")
        wid = s * _NCORES + c
        base = jnp.full((16,), wid // WPB * _NC, jnp.int32)

        def load_idx(ch, idx_v):
            blk = wid * NCH + ch
            pltpu.sync_copy(a2c_hbm.at[blk], idx_v)
            for j in range(CH // 16):
                sl = pl.ds(j * 16, 16)
                idx_v[sl] = idx_v[sl] + base

        load_idx(0, idx_a)
        pltpu.async_copy(tab_hbm.at[idx_a], rows_a, sem_a)

        def body(gg, _):
            g0 = gg * 2
            pltpu.make_async_copy(tab_hbm.at[idx_a], rows_a, sem_a).wait()
            pltpu.async_copy(rows_a, out_hbm.at[wid * NCH + g0], sem_oa)
            @pl.when(gg > 0)
            def _():
                pltpu.make_async_copy(rows_b, out_hbm.at[0], sem_ob).wait()
            load_idx(g0 + 1, idx_b)
            pltpu.async_copy(tab_hbm.at[idx_b], rows_b, sem_b)
            @pl.when(gg < NCH // 2 - 1)
            def _():
                pltpu.make_async_copy(rows_a, out_hbm.at[0], sem_oa).wait()
                load_idx(g0 + 2, idx_a)
                pltpu.async_copy(tab_hbm.at[idx_a], rows_a, sem_a)
            pltpu.make_async_copy(tab_hbm.at[idx_b], rows_b, sem_b).wait()
            pltpu.async_copy(rows_b, out_hbm.at[wid * NCH + g0 + 1], sem_ob)
            return 0
        lax.fori_loop(0, NCH // 2, body, 0)
        pltpu.make_async_copy(rows_a, out_hbm.at[0], sem_oa).wait()
        pltpu.make_async_copy(rows_b, out_hbm.at[0], sem_ob).wait()

    return gather, CH


_G_CH = 32


def _gather_call(tab, a2c_g):
    return _make_gather(_H)[0](tab, a2c_g)


def _ln(x, g, b):
    m = jnp.mean(x, axis=-1, keepdims=True)
    v = jnp.mean((x - m) ** 2, axis=-1, keepdims=True)
    return (x - m) / jnp.sqrt(v + 1e-5) * g + b


def _sigmoid(x):
    return 1.0 / (1.0 + jnp.exp(-x))


_BF16 = jnp.bfloat16


def _bdot(x, w):
    return jnp.dot(x.astype(_BF16), w, preferred_element_type=_F32)


def _coarse_body(f_ref, a_ref, g_ref, co_ref, ws_ref, vec_ref,
                 out_ref, tab_ref):
    b_f2c, g1, be1 = vec_ref[0:1], vec_ref[1:2], vec_ref[2:3]
    b_c2f, g2, be2 = vec_ref[3:4], vec_ref[4:5], vec_ref[5:6]
    b_gate, b_gi = vec_ref[6:7], vec_ref[7:8]
    g3, be3 = vec_ref[8:9], vec_ref[9:10]
    w_f2c, w_c2f = ws_ref[0], ws_ref[1]
    wg1, wg2, wgi1, wgi2 = ws_ref[2], ws_ref[3], ws_ref[4], ws_ref[5]

    idx = a_ref[0, 0]
    iota = lax.broadcasted_iota(jnp.int32, (_N, _NC), 1)
    hot = (idx[:, None] == iota)
    cnt = jnp.sum(hot.astype(_F32), axis=0)
    rec = 1.0 / jnp.maximum(cnt, 1.0)
    sums = lax.dot_general(hot.astype(jnp.bfloat16),
                           f_ref[0].astype(jnp.bfloat16),
                           (((0,), (0,)), ((), ())),
                           preferred_element_type=_F32)
    seg_mean = sums * rec[:, None]
    gs = jnp.sum(g_ref[0], axis=0, keepdims=True)

    coarse = co_ref[0]
    cff = jnp.maximum(_ln(_bdot(seg_mean, w_f2c) + b_f2c, g1, be1), 0.0)
    c2 = jnp.maximum(_ln(_bdot(coarse, w_c2f) + b_c2f, g2, be2), 0.0)
    c3 = _bdot(c2, wg2)
    cg = _sigmoid(_bdot(coarse, wg1) + _bdot(cff, wg2) + b_gate)
    cu = cg * coarse + (1.0 - cg) * cff
    gc = gs * (1.0 / _N)
    cwg = jnp.maximum(
        _ln(_bdot(cu, wgi1) + _bdot(gc, wgi2) + b_gi, g3, be3), 0.0)
    out_ref[0] = cu + 0.1 * cwg
    lob = lax.bitcast_convert_type(c2, jnp.int32)
    lob = lob + 0x7FFF + ((lob >> 16) & 1)
    hib = lax.bitcast_convert_type(c3, jnp.int32)
    hib = hib + 0x7FFF + ((hib >> 16) & 1)
    tab_ref[0] = (hib & jnp.int32(-65536)) | lax.shift_right_logical(lob, 16)


def _fine_body(f_ref, r_ref, g_ref, wg1_ref, wgi1_ref, wgi2_ref, vec_ref,
               out_ref):
    b_gate, b_gi = vec_ref[0:1], vec_ref[1:2]
    g3, be3 = vec_ref[2:3], vec_ref[3:4]
    f = f_ref[...]
    w = r_ref[...]
    c2g = lax.bitcast_convert_type(lax.shift_left(w, 16), _F32)
    g3row = lax.bitcast_convert_type(w & jnp.int32(-65536), _F32)
    fg = _sigmoid(_bdot(f, wg1_ref[...]) + g3row + b_gate)
    fu = fg * f + (1.0 - fg) * c2g
    fwg = jnp.maximum(
        _ln(_bdot(fu, wgi1_ref[...]) + _bdot(g_ref[...], wgi2_ref[...])
            + b_gi, g3, be3), 0.0)
    out_ref[...] = fu + 0.1 * fwg


def kernel(fine_features, coarse_features, atom_to_coarse, global_features,
           W_f2c, b_f2c, g1, be1, W_c2f, b_c2f, g2, be2,
           W_gate, b_gate, W_gi, b_gi, g3, be3):
    B, N, NC, H = _B, _N, _NC, _H
    f32 = _F32

    wg1, wg2 = W_gate[:H], W_gate[H:]
    wgi1, wgi2 = W_gi[:H], W_gi[H:]

    vecs = jnp.stack([b_f2c, g1, be1, b_c2f, g2, be2, b_gate, b_gi, g3, be3])
    ws = jnp.stack([W_f2c, W_c2f, wg1, wg2, wgi1, wgi2]).astype(jnp.bfloat16)
    coarse_out, tab = pl.pallas_call(
        _coarse_body,
        grid=(B,),
        in_specs=[
            pl.BlockSpec((1, N, H), lambda b: (b, 0, 0)),
            pl.BlockSpec((1, 1, N), lambda b: (b, 0, 0)),
            pl.BlockSpec((1, N, H), lambda b: (b, 0, 0)),
            pl.BlockSpec((1, NC, H), lambda b: (b, 0, 0)),
            pl.BlockSpec((6, H, H), lambda b: (0, 0, 0)),
            pl.BlockSpec((10, H), lambda b: (0, 0)),
        ],
        out_specs=[
            pl.BlockSpec((1, NC, H), lambda b: (b, 0, 0)),
            pl.BlockSpec((1, NC, H), lambda b: (b, 0, 0)),
        ],
        out_shape=[
            jax.ShapeDtypeStruct((B, NC, H), f32),
            jax.ShapeDtypeStruct((B, NC, H), jnp.int32),
        ],
    )(fine_features, atom_to_coarse.reshape(B, 1, N), global_features,
      coarse_features, ws, vecs)

    a2c_g = atom_to_coarse.reshape(B * N // _G_CH, _G_CH)
    ctx = _gather_call(tab.reshape(B * NC, H), a2c_g)
    ctx = ctx.reshape(B * N, H)

    RB = 1024
    vec4 = jnp.stack([b_gate, b_gi, g3, be3])
    fine_out = pl.pallas_call(
        _fine_body,
        grid=(B * N // RB,),
        in_specs=[
            pl.BlockSpec((RB, H), lambda i: (i, 0)),
            pl.BlockSpec((RB, H), lambda i: (i, 0)),
            pl.BlockSpec((RB, H), lambda i: (i, 0)),
            pl.BlockSpec((H, H), lambda i: (0, 0)),
            pl.BlockSpec((H, H), lambda i: (0, 0)),
            pl.BlockSpec((H, H), lambda i: (0, 0)),
            pl.BlockSpec((4, H), lambda i: (0, 0)),
        ],
        out_specs=pl.BlockSpec((RB, H), lambda i: (i, 0)),
        out_shape=jax.ShapeDtypeStruct((B * N, H), f32),
    )(fine_features.reshape(B * N, H), ctx,
      global_features.reshape(B * N, H),
      wg1.astype(jnp.bfloat16), wgi1.astype(jnp.bfloat16),
      wgi2.astype(jnp.bfloat16), vec4)

    return fine_out.reshape(B, N, H), coarse_out


# speedup vs baseline: 5.2837x; 1.0152x over previous
import functools

import jax
import jax.numpy as jnp
from jax import lax
from jax.experimental import pallas as pl
from jax.experimental.pallas import tpu as pltpu
from jax.experimental.pallas import tpu_sc as plsc

_B, _N, _NC, _H = 8, 4096, 512, 512
_F32 = jnp.float32

_NCORES = 2
_NSUB = 16


@functools.lru_cache(maxsize=None)
def _make_gather(W):
    NW = _NCORES * _NSUB
    APT = _B * _N // NW
    CH = 64
    NCH = APT // CH
    WPB = NW // _B
    mesh = plsc.VectorSubcoreMesh(core_axis_name="c", subcore_axis_name="s",
                                  num_cores=_NCORES, num_subcores=_NSUB)

    @functools.partial(
        pl.kernel,
        out_type=jax.ShapeDtypeStruct((_B * _N // CH, CH, W), jnp.int32),
        mesh=mesh,
    scratch_types=[
        pltpu.VMEM((CH,), jnp.int32),
        pltpu.VMEM((CH,), jnp.int32),
        pltpu.VMEM((CH, W), jnp.int32),
        pltpu.VMEM((CH, W), jnp.int32),
        pltpu.SemaphoreType.DMA,
        pltpu.SemaphoreType.DMA,
        pltpu.SemaphoreType.DMA,
        pltpu.SemaphoreType.DMA,
    ],
    )
    def gather(tab_hbm, a2c_hbm, out_hbm, idx_a, idx_b, rows_a, rows_b,
               sem_a, sem_b, sem_oa, sem_ob):
        c = lax.axis_index("c")
        s = lax.axis_index("s")
        wid = s * _NCORES + c
        base = jnp.full((16,), wid // WPB * _NC, jnp.int32)

        def load_idx(ch, idx_v):
            blk = wid * NCH + ch
            pltpu.sync_copy(a2c_hbm.at[blk], idx_v)
            for j in range(CH // 16):
                sl = pl.ds(j * 16, 16)
                idx_v[sl] = idx_v[sl] + base

        load_idx(0, idx_a)
        pltpu.async_copy(tab_hbm.at[idx_a], rows_a, sem_a)

        def body(gg, _):
            g0 = gg * 2
            pltpu.make_async_copy(tab_hbm.at[idx_a], rows_a, sem_a).wait()
            pltpu.async_copy(rows_a, out_hbm.at[wid * NCH + g0], sem_oa)
            @pl.when(gg > 0)
            def _():
                pltpu.make_async_copy(rows_b, out_hbm.at[0], sem_ob).wait()
            load_idx(g0 + 1, idx_b)
            pltpu.async_copy(tab_hbm.at[idx_b], rows_b, sem_b)
            @pl.when(gg < NCH // 2 - 1)
            def _():
                pltpu.make_async_copy(rows_a, out_hbm.at[0], sem_oa).wait()
                load_idx(g0 + 2, idx_a)
                pltpu.async_copy(tab_hbm.at[idx_a], rows_a, sem_a)
            pltpu.make_async_copy(tab_hbm.at[idx_b], rows_b, sem_b).wait()
            pltpu.async_copy(rows_b, out_hbm.at[wid * NCH + g0 + 1], sem_ob)
            return 0
        lax.fori_loop(0, NCH // 2, body, 0)
        pltpu.make_async_copy(rows_a, out_hbm.at[0], sem_oa).wait()
        pltpu.make_async_copy(rows_b, out_hbm.at[0], sem_ob).wait()

    return gather, CH


_G_CH = 64


def _gather_call(tab, a2c_g):
    return _make_gather(_H)[0](tab, a2c_g)


def _ln(x, g, b):
    m = jnp.mean(x, axis=-1, keepdims=True)
    v = jnp.mean((x - m) ** 2, axis=-1, keepdims=True)
    return (x - m) / jnp.sqrt(v + 1e-5) * g + b


def _sigmoid(x):
    return 1.0 / (1.0 + jnp.exp(-x))


_BF16 = jnp.bfloat16


def _bdot(x, w):
    return jnp.dot(x.astype(_BF16), w, preferred_element_type=_F32)


def _coarse_body(f_ref, a_ref, g_ref, co_ref, ws_ref, vec_ref,
                 out_ref, tab_ref):
    b_f2c, g1, be1 = vec_ref[0:1], vec_ref[1:2], vec_ref[2:3]
    b_c2f, g2, be2 = vec_ref[3:4], vec_ref[4:5], vec_ref[5:6]
    b_gate, b_gi = vec_ref[6:7], vec_ref[7:8]
    g3, be3 = vec_ref[8:9], vec_ref[9:10]
    w_f2c, w_c2f = ws_ref[0], ws_ref[1]
    wg1, wg2, wgi1, wgi2 = ws_ref[2], ws_ref[3], ws_ref[4], ws_ref[5]

    idx = a_ref[0, 0]
    iota = lax.broadcasted_iota(jnp.int32, (_N, _NC), 1)
    hot = (idx[:, None] == iota)
    cnt = jnp.sum(hot.astype(_F32), axis=0)
    rec = 1.0 / jnp.maximum(cnt, 1.0)
    sums = lax.dot_general(hot.astype(jnp.bfloat16),
                           f_ref[0].astype(jnp.bfloat16),
                           (((0,), (0,)), ((), ())),
                           preferred_element_type=_F32)
    seg_mean = sums * rec[:, None]
    gs = jnp.sum(g_ref[0], axis=0, keepdims=True)

    coarse = co_ref[0]
    cff = jnp.maximum(_ln(_bdot(seg_mean, w_f2c) + b_f2c, g1, be1), 0.0)
    c2 = jnp.maximum(_ln(_bdot(coarse, w_c2f) + b_c2f, g2, be2), 0.0)
    c3 = _bdot(c2, wg2)
    cg = _sigmoid(_bdot(coarse, wg1) + _bdot(cff, wg2) + b_gate)
    cu = cg * coarse + (1.0 - cg) * cff
    gc = gs * (1.0 / _N)
    cwg = jnp.maximum(
        _ln(_bdot(cu, wgi1) + _bdot(gc, wgi2) + b_gi, g3, be3), 0.0)
    out_ref[0] = cu + 0.1 * cwg
    lob = lax.bitcast_convert_type(c2, jnp.int32)
    lob = lob + 0x7FFF + ((lob >> 16) & 1)
    hib = lax.bitcast_convert_type(c3, jnp.int32)
    hib = hib + 0x7FFF + ((hib >> 16) & 1)
    tab_ref[0] = (hib & jnp.int32(-65536)) | lax.shift_right_logical(lob, 16)


def _fine_body(f_ref, r_ref, g_ref, wg1_ref, wgi1_ref, wgi2_ref, vec_ref,
               out_ref):
    b_gate, b_gi = vec_ref[0:1], vec_ref[1:2]
    g3, be3 = vec_ref[2:3], vec_ref[3:4]
    f = f_ref[...]
    w = r_ref[...]
    c2g = lax.bitcast_convert_type(lax.shift_left(w, 16), _F32)
    g3row = lax.bitcast_convert_type(w & jnp.int32(-65536), _F32)
    fg = _sigmoid(_bdot(f, wg1_ref[...]) + g3row + b_gate)
    fu = fg * f + (1.0 - fg) * c2g
    fwg = jnp.maximum(
        _ln(_bdot(fu, wgi1_ref[...]) + _bdot(g_ref[...], wgi2_ref[...])
            + b_gi, g3, be3), 0.0)
    out_ref[...] = fu + 0.1 * fwg


def kernel(fine_features, coarse_features, atom_to_coarse, global_features,
           W_f2c, b_f2c, g1, be1, W_c2f, b_c2f, g2, be2,
           W_gate, b_gate, W_gi, b_gi, g3, be3):
    B, N, NC, H = _B, _N, _NC, _H
    f32 = _F32

    wg1, wg2 = W_gate[:H], W_gate[H:]
    wgi1, wgi2 = W_gi[:H], W_gi[H:]

    vecs = jnp.stack([b_f2c, g1, be1, b_c2f, g2, be2, b_gate, b_gi, g3, be3])
    ws = jnp.stack([W_f2c, W_c2f, wg1, wg2, wgi1, wgi2]).astype(jnp.bfloat16)
    coarse_out, tab = pl.pallas_call(
        _coarse_body,
        grid=(B,),
        in_specs=[
            pl.BlockSpec((1, N, H), lambda b: (b, 0, 0)),
            pl.BlockSpec((1, 1, N), lambda b: (b, 0, 0)),
            pl.BlockSpec((1, N, H), lambda b: (b, 0, 0)),
            pl.BlockSpec((1, NC, H), lambda b: (b, 0, 0)),
            pl.BlockSpec((6, H, H), lambda b: (0, 0, 0)),
            pl.BlockSpec((10, H), lambda b: (0, 0)),
        ],
        out_specs=[
            pl.BlockSpec((1, NC, H), lambda b: (b, 0, 0)),
            pl.BlockSpec((1, NC, H), lambda b: (b, 0, 0)),
        ],
        out_shape=[
            jax.ShapeDtypeStruct((B, NC, H), f32),
            jax.ShapeDtypeStruct((B, NC, H), jnp.int32),
        ],
    )(fine_features, atom_to_coarse.reshape(B, 1, N), global_features,
      coarse_features, ws, vecs)

    a2c_g = atom_to_coarse.reshape(B * N // _G_CH, _G_CH)
    ctx = _gather_call(tab.reshape(B * NC, H), a2c_g)
    ctx = ctx.reshape(B * N, H)

    RB = 1024
    vec4 = jnp.stack([b_gate, b_gi, g3, be3])
    fine_out = pl.pallas_call(
        _fine_body,
        grid=(B * N // RB,),
        in_specs=[
            pl.BlockSpec((RB, H), lambda i: (i, 0)),
            pl.BlockSpec((RB, H), lambda i: (i, 0)),
            pl.BlockSpec((RB, H), lambda i: (i, 0)),
            pl.BlockSpec((H, H), lambda i: (0, 0)),
            pl.BlockSpec((H, H), lambda i: (0, 0)),
            pl.BlockSpec((H, H), lambda i: (0, 0)),
            pl.BlockSpec((4, H), lambda i: (0, 0)),
        ],
        out_specs=pl.BlockSpec((RB, H), lambda i: (i, 0)),
        out_shape=jax.ShapeDtypeStruct((B * N, H), f32),
    )(fine_features.reshape(B * N, H), ctx,
      global_features.reshape(B * N, H),
      wg1.astype(jnp.bfloat16), wgi1.astype(jnp.bfloat16),
      wgi2.astype(jnp.bfloat16), vec4)

    return fine_out.reshape(B, N, H), coarse_out


# speedup vs baseline: 5.4881x; 1.0387x over previous
import functools

import jax
import jax.numpy as jnp
from jax import lax
from jax.experimental import pallas as pl
from jax.experimental.pallas import tpu as pltpu
from jax.experimental.pallas import tpu_sc as plsc

_B, _N, _NC, _H = 8, 4096, 512, 512
_F32 = jnp.float32

_NCORES = 2
_NSUB = 16


@functools.lru_cache(maxsize=None)
def _make_gather(W):
    NW = _NCORES * _NSUB
    APT = _B * _N // NW
    CH = 64
    NCH = APT // CH
    WPB = NW // _B
    mesh = plsc.VectorSubcoreMesh(core_axis_name="c", subcore_axis_name="s",
                                  num_cores=_NCORES, num_subcores=_NSUB)

    @functools.partial(
        pl.kernel,
        out_type=jax.ShapeDtypeStruct((_B * _N // CH, CH, W), jnp.int32),
        mesh=mesh,
    scratch_types=[
        pltpu.VMEM((CH,), jnp.int32),
        pltpu.VMEM((CH,), jnp.int32),
        pltpu.VMEM((CH, W), jnp.int32),
        pltpu.VMEM((CH, W), jnp.int32),
        pltpu.SemaphoreType.DMA,
        pltpu.SemaphoreType.DMA,
        pltpu.SemaphoreType.DMA,
        pltpu.SemaphoreType.DMA,
    ],
    )
    def gather(tab_hbm, a2c_hbm, out_hbm, idx_a, idx_b, rows_a, rows_b,
               sem_a, sem_b, sem_oa, sem_ob):
        c = lax.axis_index("c")
        s = lax.axis_index("s")
        wid = s * _NCORES + c
        base = jnp.full((16,), wid // WPB * _NC, jnp.int32)

        def load_idx(ch, idx_v):
            blk = wid * NCH + ch
            pltpu.sync_copy(a2c_hbm.at[blk], idx_v)
            for j in range(CH // 16):
                sl = pl.ds(j * 16, 16)
                idx_v[sl] = idx_v[sl] + base

        load_idx(0, idx_a)
        pltpu.async_copy(tab_hbm.at[idx_a], rows_a, sem_a)

        def body(gg, _):
            g0 = gg * 2
            pltpu.make_async_copy(tab_hbm.at[idx_a], rows_a, sem_a).wait()
            pltpu.async_copy(rows_a, out_hbm.at[wid * NCH + g0], sem_oa)
            @pl.when(gg > 0)
            def _():
                pltpu.make_async_copy(rows_b, out_hbm.at[0], sem_ob).wait()
            load_idx(g0 + 1, idx_b)
            pltpu.async_copy(tab_hbm.at[idx_b], rows_b, sem_b)
            @pl.when(gg < NCH // 2 - 1)
            def _():
                pltpu.make_async_copy(rows_a, out_hbm.at[0], sem_oa).wait()
                load_idx(g0 + 2, idx_a)
                pltpu.async_copy(tab_hbm.at[idx_a], rows_a, sem_a)
            pltpu.make_async_copy(tab_hbm.at[idx_b], rows_b, sem_b).wait()
            pltpu.async_copy(rows_b, out_hbm.at[wid * NCH + g0 + 1], sem_ob)
            return 0
        lax.fori_loop(0, NCH // 2, body, 0)
        pltpu.make_async_copy(rows_a, out_hbm.at[0], sem_oa).wait()
        pltpu.make_async_copy(rows_b, out_hbm.at[0], sem_ob).wait()

    return gather, CH


_G_CH = 64


def _gather_call(tab, a2c_g):
    return _make_gather(_H)[0](tab, a2c_g)


def _ln(x, g, b):
    m = jnp.mean(x, axis=-1, keepdims=True)
    v = jnp.mean((x - m) ** 2, axis=-1, keepdims=True)
    return (x - m) / jnp.sqrt(v + 1e-5) * g + b


def _sigmoid(x):
    return 1.0 / (1.0 + jnp.exp(-x))


_BF16 = jnp.bfloat16


def _bdot(x, w):
    return jnp.dot(x.astype(_BF16), w.astype(_BF16),
                   preferred_element_type=_F32)


def _coarse_body(f_ref, a_ref, g_ref, co_ref, wf_ref, wc_ref, wgt_ref,
                 wgi_ref, vec_ref, out_ref, tab_ref):
    b_f2c, g1, be1 = vec_ref[0:1], vec_ref[1:2], vec_ref[2:3]
    b_c2f, g2, be2 = vec_ref[3:4], vec_ref[4:5], vec_ref[5:6]
    b_gate, b_gi = vec_ref[6:7], vec_ref[7:8]
    g3, be3 = vec_ref[8:9], vec_ref[9:10]
    w_f2c, w_c2f = wf_ref[...], wc_ref[...]
    wg1, wg2 = wgt_ref[:_H], wgt_ref[_H:]
    wgi1, wgi2 = wgi_ref[:_H], wgi_ref[_H:]

    idx = a_ref[0, 0]
    iota = lax.broadcasted_iota(jnp.int32, (_N, _NC), 1)
    hot = (idx[:, None] == iota)
    cnt = jnp.sum(hot.astype(_F32), axis=0)
    rec = 1.0 / jnp.maximum(cnt, 1.0)
    sums = lax.dot_general(hot.astype(jnp.bfloat16),
                           f_ref[0].astype(jnp.bfloat16),
                           (((0,), (0,)), ((), ())),
                           preferred_element_type=_F32)
    seg_mean = sums * rec[:, None]
    gs = jnp.sum(g_ref[0], axis=0, keepdims=True)

    coarse = co_ref[0]
    cff = jnp.maximum(_ln(_bdot(seg_mean, w_f2c) + b_f2c, g1, be1), 0.0)
    c2 = jnp.maximum(_ln(_bdot(coarse, w_c2f) + b_c2f, g2, be2), 0.0)
    c3 = _bdot(c2, wg2)
    cg = _sigmoid(_bdot(coarse, wg1) + _bdot(cff, wg2) + b_gate)
    cu = cg * coarse + (1.0 - cg) * cff
    gc = gs * (1.0 / _N)
    cwg = jnp.maximum(
        _ln(_bdot(cu, wgi1) + _bdot(gc, wgi2) + b_gi, g3, be3), 0.0)
    out_ref[0] = cu + 0.1 * cwg
    lob = lax.bitcast_convert_type(c2, jnp.int32)
    lob = lob + 0x7FFF + ((lob >> 16) & 1)
    hib = lax.bitcast_convert_type(c3, jnp.int32)
    hib = hib + 0x7FFF + ((hib >> 16) & 1)
    tab_ref[0] = (hib & jnp.int32(-65536)) | lax.shift_right_logical(lob, 16)


def _fine_body(f_ref, r_ref, g_ref, wgt_ref, wgi_ref, vec_ref, out_ref):
    b_gate, b_gi = vec_ref[0:1], vec_ref[1:2]
    g3, be3 = vec_ref[2:3], vec_ref[3:4]
    f = f_ref[...]
    w = r_ref[...]
    c2g = lax.bitcast_convert_type(lax.shift_left(w, 16), _F32)
    g3row = lax.bitcast_convert_type(w & jnp.int32(-65536), _F32)
    fg = _sigmoid(_bdot(f, wgt_ref[:_H]) + g3row + b_gate)
    fu = fg * f + (1.0 - fg) * c2g
    fwg = jnp.maximum(
        _ln(_bdot(fu, wgi_ref[:_H]) + _bdot(g_ref[...], wgi_ref[_H:])
            + b_gi, g3, be3), 0.0)
    out_ref[...] = fu + 0.1 * fwg


def kernel(fine_features, coarse_features, atom_to_coarse, global_features,
           W_f2c, b_f2c, g1, be1, W_c2f, b_c2f, g2, be2,
           W_gate, b_gate, W_gi, b_gi, g3, be3):
    B, N, NC, H = _B, _N, _NC, _H
    f32 = _F32

    vecs = jnp.stack([b_f2c, g1, be1, b_c2f, g2, be2, b_gate, b_gi, g3, be3])
    coarse_out, tab = pl.pallas_call(
        _coarse_body,
        grid=(B,),
        in_specs=[
            pl.BlockSpec((1, N, H), lambda b: (b, 0, 0)),
            pl.BlockSpec((1, 1, N), lambda b: (b, 0, 0)),
            pl.BlockSpec((1, N, H), lambda b: (b, 0, 0)),
            pl.BlockSpec((1, NC, H), lambda b: (b, 0, 0)),
            pl.BlockSpec((H, H), lambda b: (0, 0)),
            pl.BlockSpec((H, H), lambda b: (0, 0)),
            pl.BlockSpec((2 * H, H), lambda b: (0, 0)),
            pl.BlockSpec((2 * H, H), lambda b: (0, 0)),
            pl.BlockSpec((10, H), lambda b: (0, 0)),
        ],
        out_specs=[
            pl.BlockSpec((1, NC, H), lambda b: (b, 0, 0)),
            pl.BlockSpec((1, NC, H), lambda b: (b, 0, 0)),
        ],
        out_shape=[
            jax.ShapeDtypeStruct((B, NC, H), f32),
            jax.ShapeDtypeStruct((B, NC, H), jnp.int32),
        ],
    )(fine_features, atom_to_coarse.reshape(B, 1, N), global_features,
      coarse_features, W_f2c, W_c2f, W_gate, W_gi, vecs)

    a2c_g = atom_to_coarse.reshape(B * N // _G_CH, _G_CH)
    ctx = _gather_call(tab.reshape(B * NC, H), a2c_g)
    ctx = ctx.reshape(B * N, H)

    RB = 2048
    vec4 = jnp.stack([b_gate, b_gi, g3, be3])
    fine_out = pl.pallas_call(
        _fine_body,
        grid=(B * N // RB,),
        in_specs=[
            pl.BlockSpec((RB, H), lambda i: (i, 0)),
            pl.BlockSpec((RB, H), lambda i: (i, 0)),
            pl.BlockSpec((RB, H), lambda i: (i, 0)),
            pl.BlockSpec((2 * H, H), lambda i: (0, 0)),
            pl.BlockSpec((2 * H, H), lambda i: (0, 0)),
            pl.BlockSpec((4, H), lambda i: (0, 0)),
        ],
        out_specs=pl.BlockSpec((RB, H), lambda i: (i, 0)),
        out_shape=jax.ShapeDtypeStruct((B * N, H), f32),
    )(fine_features.reshape(B * N, H), ctx,
      global_features.reshape(B * N, H), W_gate, W_gi, vec4)

    return fine_out.reshape(B, N, H), coarse_out
